# Initial kernel scaffold; baseline (speedup 1.0000x reference)
import jax
import jax.numpy as jnp
from jax.experimental import pallas as pl


def kernel(raw):
    raise NotImplementedError("write your pallas kernel here")


# speedup vs baseline: 3.0766x; 3.0766x over previous
import functools

import jax
import jax.numpy as jnp
from jax.experimental import pallas as pl
from jax.experimental.pallas import tpu as pltpu

STRIDE = 8.0
GRID = 128
N = GRID * GRID
B = 4
MAX_DET = 100
CONF_TH = 0.25
IOU_TH = 0.45


def _nms_kernel(r0, r1, r2, r3, r4, r5, ocx, ocy, ow, oh, oconf,
                x1s, y1s, x2s, y2s, areas, cxs, cys, ws, hs, confs, scs):
    lane = jax.lax.broadcasted_iota(jnp.int32, (B, N), 1)
    gx = (lane % GRID).astype(jnp.float32)
    gy = (lane // GRID).astype(jnp.float32)
    cx = (jax.nn.sigmoid(r0[...]) * 2.0 - 0.5 + gx) * STRIDE
    cy = (jax.nn.sigmoid(r1[...]) * 2.0 - 0.5 + gy) * STRIDE
    w = (jax.nn.sigmoid(r2[...]) * 2.0) ** 2 * (STRIDE * 4.0)
    h = (jax.nn.sigmoid(r3[...]) * 2.0) ** 2 * (STRIDE * 4.0)
    conf = jax.nn.sigmoid(r4[...]) * jax.nn.sigmoid(r5[...])

    x1 = cx - w * 0.5
    y1 = cy - h * 0.5
    x2 = cx + w * 0.5
    y2 = cy + h * 0.5
    area = jnp.maximum(x2 - x1, 0.0) * jnp.maximum(y2 - y1, 0.0)

    zeros_out = jnp.zeros((B, 128), jnp.float32)
    ocx[...] = zeros_out
    ocy[...] = zeros_out
    ow[...] = zeros_out
    oh[...] = zeros_out
    oconf[...] = zeros_out
    lane_out = jax.lax.broadcasted_iota(jnp.int32, (B, 128), 1)

    x1s[...] = x1
    y1s[...] = y1
    x2s[...] = x2
    y2s[...] = y2
    areas[...] = area
    cxs[...] = cx
    cys[...] = cy
    ws[...] = w
    hs[...] = h
    confs[...] = conf
    scs[...] = jnp.where(conf >= CONF_TH, conf, -1.0)

    def step(t, _):
        scores = scs[...]
        m = jnp.max(scores, axis=1, keepdims=True)
        valid = m > 0.0
        hit = scores == m
        win = jnp.min(jnp.where(hit, lane, N), axis=1, keepdims=True)
        onehot = lane == win

        def gather(ref):
            return jnp.sum(jnp.where(onehot, ref[...], 0.0), axis=1,
                           keepdims=True)

        wcx, wcy, ww, wh, wconf = (gather(cxs), gather(cys), gather(ws),
                                   gather(hs), gather(confs))
        wx1 = wcx - ww * 0.5
        wy1 = wcy - wh * 0.5
        wx2 = wcx + ww * 0.5
        wy2 = wcy + wh * 0.5
        warea = ww * wh

        xx1 = jnp.maximum(wx1, x1s[...])
        yy1 = jnp.maximum(wy1, y1s[...])
        xx2 = jnp.minimum(wx2, x2s[...])
        yy2 = jnp.minimum(wy2, y2s[...])
        inter = jnp.maximum(xx2 - xx1, 0.0) * jnp.maximum(yy2 - yy1, 0.0)
        iou = inter / (warea + areas[...] - inter + 1e-9)
        kill = (iou > IOU_TH) | onehot
        scs[...] = jnp.where(valid & kill, -1.0, scores)

        slot = lane_out == t
        def put(ref, val):
            ref[...] = jnp.where(slot & valid, val, ref[...])
        put(ocx, wcx)
        put(ocy, wcy)
        put(ow, ww)
        put(oh, wh)
        put(oconf, wconf)
        return ()

    jax.lax.fori_loop(0, MAX_DET, step, (), unroll=False)


@jax.jit
def kernel(raw):
    out_shape = [jax.ShapeDtypeStruct((B, 128), jnp.float32)] * 5
    scratch = [pltpu.VMEM((B, N), jnp.float32)] * 11
    raw_t = jnp.transpose(raw, (2, 0, 1))
    ocx, ocy, ow, oh, oconf = pl.pallas_call(
        _nms_kernel,
        out_shape=out_shape,
        scratch_shapes=scratch,
    )(raw_t[0], raw_t[1], raw_t[2], raw_t[3], raw_t[4], raw_t[5])
    cls = jnp.zeros_like(oconf)
    out = jnp.stack([ocx, ocy, ow, oh, oconf, cls], axis=-1)
    return out[:, :MAX_DET, :]
